# baseline (speedup 1.0000x reference)
import functools

import jax
import jax.numpy as jnp
from jax import lax
from jax.experimental import pallas as pl
from jax.experimental.pallas import tpu as pltpu
from jax.experimental.pallas import tpu_sc as plsc

BATCH = 16384
FACTOR = 32
NC = 2
NS = 16
NW = NC * NS
TC_N = 4096
SPLIT = BATCH - TC_N
BPW = SPLIT // NW
RING = 8
NGRP = BPW // 16
TC_RING = 16
TC_CHUNK = 1024

_mesh = plsc.VectorSubcoreMesh(core_axis_name="c", subcore_axis_name="s")


@functools.partial(
    pl.kernel,
    out_type=jax.ShapeDtypeStruct((SPLIT,), jnp.float32),
    mesh=_mesh,
    compiler_params=pltpu.CompilerParams(
        needs_layout_passes=False, use_tc_tiling_on_sc=True),
    scratch_types=[
        pltpu.VMEM((BPW,), jnp.int32),
        pltpu.VMEM((BPW,), jnp.int32),
        pltpu.VMEM((FACTOR, RING * 256), jnp.float32),
        pltpu.VMEM((BPW,), jnp.float32),
    ] + [pltpu.SemaphoreType.DMA] * RING,
)
def _mf_sc(user_hbm, item_hbm, uft_hbm, ift_hbm, out_hbm,
           uidx_v, iidx_v, cols_v, out_v, *sems):
    wid = lax.axis_index("s") * NC + lax.axis_index("c")
    base = wid * BPW

    pltpu.sync_copy(user_hbm.at[pl.ds(base, BPW)], uidx_v)
    pltpu.sync_copy(item_hbm.at[pl.ds(base, BPW)], iidx_v)

    lane = lax.iota(jnp.int32, 16)

    def fire(slot, iu, iv):
        cu = pl.multiple_of((iu >> 7) << 7, 128)
        ci = pl.multiple_of((iv >> 7) << 7, 128)
        pltpu.async_copy(uft_hbm.at[:, pl.ds(cu, 128)],
                         cols_v.at[:, pl.ds(slot * 256, 128)], sems[slot])
        pltpu.async_copy(ift_hbm.at[:, pl.ds(ci, 128)],
                         cols_v.at[:, pl.ds(slot * 256 + 128, 128)],
                         sems[slot])

    def wait_slot(slot):
        for _ in range(2):
            pltpu.make_async_copy(
                uft_hbm.at[:, pl.ds(0, 128)],
                cols_v.at[:, pl.ds(slot * 256, 128)], sems[slot]).wait()

    uvec0 = uidx_v[pl.ds(0, 16)]
    ivec0 = iidx_v[pl.ds(0, 16)]
    for j in range(RING):
        fire(j, uvec0[j], ivec0[j])

    def group_body(g, _):
        uvec = uidx_v[pl.ds(g * 16, 16)]
        ivec = iidx_v[pl.ds(g * 16, 16)]
        nxt = lax.rem((g + 1) * 16, BPW)
        nuvec = uidx_v[pl.ds(nxt, 16)]
        nivec = iidx_v[pl.ds(nxt, 16)]
        acc = jnp.zeros((16,), jnp.float32)
        for j in range(16):
            slot = j % RING
            wait_slot(slot)
            ucol = jnp.full((16,), slot * 256, jnp.int32) + (uvec[j] & 127)
            icol = jnp.full((16,), slot * 256 + 128, jnp.int32) + (ivec[j] & 127)
            u0 = plsc.load_gather(cols_v, [lane, ucol])
            u1 = plsc.load_gather(cols_v, [lane + 16, ucol])
            v0 = plsc.load_gather(cols_v, [lane, icol])
            v1 = plsc.load_gather(cols_v, [lane + 16, icol])
            p = u0 * v0 + u1 * v1
            s = jnp.sum(p)
            acc = jnp.where(lane == j, s, acc)
            if j < RING:
                @pl.when(g * 16 + j + RING < BPW)
                def _():
                    fire(slot, uvec[j + RING], ivec[j + RING])
            else:
                @pl.when((g + 1) * 16 + (j - RING) < BPW)
                def _():
                    fire(slot, nuvec[j - RING], nivec[j - RING])
        out_v[pl.ds(g * 16, 16)] = acc
        return 0

    lax.fori_loop(0, NGRP, group_body, 0)

    pltpu.sync_copy(out_v, out_hbm.at[pl.ds(base, BPW)])


def _tc_body(uidx_s, iidx_s, uft, ift, out_ref, cols_v, sems):
    c = pl.program_id(0)
    base = c * TC_CHUNK
    iota128 = lax.broadcasted_iota(jnp.int32, (128, 1), 0)

    def fire(k, slot):
        iu = uidx_s[base + k]
        iv = iidx_s[base + k]
        cu = pl.multiple_of((iu >> 7) << 7, 128)
        ci = pl.multiple_of((iv >> 7) << 7, 128)
        pltpu.make_async_copy(uft.at[:, pl.ds(cu, 128)],
                              cols_v.at[slot, :, pl.ds(0, 128)],
                              sems.at[slot]).start()
        pltpu.make_async_copy(ift.at[:, pl.ds(ci, 128)],
                              cols_v.at[slot, :, pl.ds(128, 128)],
                              sems.at[slot]).start()

    for k in range(TC_RING):
        fire(k, k)

    def step(k, acc):
        slot = lax.rem(k, TC_RING)
        for _ in range(2):
            pltpu.make_async_copy(uft.at[:, pl.ds(0, 128)],
                                  cols_v.at[slot, :, pl.ds(0, 128)],
                                  sems.at[slot]).wait()
        iu = uidx_s[base + k]
        iv = iidx_s[base + k]
        oh_u = jnp.where(iota128 == (iu & 127), 1.0, 0.0)
        oh_v = jnp.where(iota128 == (iv & 127), 1.0, 0.0)
        ublk = cols_v[slot, :, pl.ds(0, 128)]
        vblk = cols_v[slot, :, pl.ds(128, 128)]
        ucol = jnp.dot(ublk, oh_u, preferred_element_type=jnp.float32)
        vcol = jnp.dot(vblk, oh_v, preferred_element_type=jnp.float32)
        s = jnp.sum(ucol * vcol)
        sub = lax.broadcasted_iota(jnp.int32, (8, 128), 0)
        ln = lax.broadcasted_iota(jnp.int32, (8, 128), 1)
        hit = (sub == k // 128) & (ln == lax.rem(k, 128))
        acc = jnp.where(hit, s, acc)

        @pl.when(k + TC_RING < TC_CHUNK)
        def _():
            fire(k + TC_RING, slot)

        return acc

    acc = lax.fori_loop(0, TC_CHUNK, step, jnp.zeros((8, 128), jnp.float32))
    out_ref[...] = acc


_mf_tc = pl.pallas_call(
    _tc_body,
    grid_spec=pltpu.PrefetchScalarGridSpec(
        num_scalar_prefetch=2,
        grid=(TC_N // TC_CHUNK,),
        in_specs=[
            pl.BlockSpec(memory_space=pl.ANY),
            pl.BlockSpec(memory_space=pl.ANY),
        ],
        out_specs=pl.BlockSpec((8, 128), lambda c, u, it: (c, 0)),
        scratch_shapes=[
            pltpu.VMEM((TC_RING, FACTOR, 256), jnp.float32),
            pltpu.SemaphoreType.DMA((TC_RING,)),
        ],
    ),
    out_shape=jax.ShapeDtypeStruct((TC_N // 128, 128), jnp.float32),
)


def kernel(user, item, user_factors, item_factors):
    uft = user_factors.T
    ift = item_factors.T
    out_sc = _mf_sc(user[:SPLIT], item[:SPLIT], uft, ift)
    out_tc = _mf_tc(user[SPLIT:], item[SPLIT:], uft, ift)
    return jnp.concatenate([out_sc, out_tc.reshape(TC_N)])


# speedup vs baseline: 5.2813x; 5.2813x over previous
import functools

import jax
import jax.numpy as jnp
from jax import lax
from jax.experimental import pallas as pl
from jax.experimental.pallas import tpu as pltpu
from jax.experimental.pallas import tpu_sc as plsc

BATCH = 16384
FACTOR = 32
NC = 2
NS = 16
NW = NC * NS
BPW = BATCH // NW
RING = 8
NGRP = BPW // 16

_mesh = plsc.VectorSubcoreMesh(core_axis_name="c", subcore_axis_name="s")


@functools.partial(
    pl.kernel,
    out_type=jax.ShapeDtypeStruct((BATCH,), jnp.float32),
    mesh=_mesh,
    compiler_params=pltpu.CompilerParams(
        needs_layout_passes=False, use_tc_tiling_on_sc=True),
    scratch_types=[
        pltpu.VMEM((BPW,), jnp.int32),
        pltpu.VMEM((BPW,), jnp.int32),
        pltpu.VMEM((FACTOR, RING * 256), jnp.float32),
        pltpu.VMEM((BPW,), jnp.float32),
    ] + [pltpu.SemaphoreType.DMA] * RING,
)
def _mf_kernel(user_hbm, item_hbm, uft_hbm, ift_hbm, out_hbm,
               uidx_v, iidx_v, cols_v, out_v, *sems):
    wid = lax.axis_index("s") * NC + lax.axis_index("c")
    base = wid * BPW

    pltpu.sync_copy(user_hbm.at[pl.ds(base, BPW)], uidx_v)
    pltpu.sync_copy(item_hbm.at[pl.ds(base, BPW)], iidx_v)

    lane = lax.iota(jnp.int32, 16)

    def fire(slot, iu, iv):
        cu = pl.multiple_of((iu >> 7) << 7, 128)
        ci = pl.multiple_of((iv >> 7) << 7, 128)
        pltpu.async_copy(uft_hbm.at[:, pl.ds(cu, 128)],
                         cols_v.at[:, pl.ds(slot * 256, 128)], sems[slot])
        pltpu.async_copy(ift_hbm.at[:, pl.ds(ci, 128)],
                         cols_v.at[:, pl.ds(slot * 256 + 128, 128)],
                         sems[slot])

    def wait_slot(slot):
        for _ in range(2):
            pltpu.make_async_copy(
                uft_hbm.at[:, pl.ds(0, 128)],
                cols_v.at[:, pl.ds(slot * 256, 128)], sems[slot]).wait()

    uvec0 = uidx_v[pl.ds(0, 16)]
    ivec0 = iidx_v[pl.ds(0, 16)]
    for j in range(RING):
        fire(j, uvec0[j], ivec0[j])

    def group_body(g, _):
        uvec = uidx_v[pl.ds(g * 16, 16)]
        ivec = iidx_v[pl.ds(g * 16, 16)]
        nxt = lax.rem((g + 1) * 16, BPW)
        nuvec = uidx_v[pl.ds(nxt, 16)]
        nivec = iidx_v[pl.ds(nxt, 16)]
        acc = jnp.zeros((16,), jnp.float32)
        for j in range(16):
            slot = j % RING
            wait_slot(slot)
            ucol = jnp.full((16,), slot * 256, jnp.int32) + (uvec[j] & 127)
            icol = jnp.full((16,), slot * 256 + 128, jnp.int32) + (ivec[j] & 127)
            u0 = plsc.load_gather(cols_v, [lane, ucol])
            u1 = plsc.load_gather(cols_v, [lane + 16, ucol])
            v0 = plsc.load_gather(cols_v, [lane, icol])
            v1 = plsc.load_gather(cols_v, [lane + 16, icol])
            p = u0 * v0 + u1 * v1
            s = jnp.sum(p)
            acc = jnp.where(lane == j, s, acc)
            if j < RING:
                @pl.when(g * 16 + j + RING < BPW)
                def _():
                    fire(slot, uvec[j + RING], ivec[j + RING])
            else:
                @pl.when((g + 1) * 16 + (j - RING) < BPW)
                def _():
                    fire(slot, nuvec[j - RING], nivec[j - RING])
        out_v[pl.ds(g * 16, 16)] = acc
        return 0

    lax.fori_loop(0, NGRP, group_body, 0)

    pltpu.sync_copy(out_v, out_hbm.at[pl.ds(base, BPW)])


def kernel(user, item, user_factors, item_factors):
    return _mf_kernel(user, item, user_factors.T, item_factors.T)
